# baseline (speedup 1.0000x reference)
import jax
import jax.numpy as jnp
from jax.experimental import pallas as pl
from jax.experimental.pallas import tpu as pltpu

_B, _H, _W, _C = 32, 16, 16, 64
_N = _H * _W
_R = _B * _N
_E = _N * (_N - 1)
_G = 2
_STEPS = _B // _G


def _build_kernel(x_ref, edges_ref, nodes_ref, batch_ref, tmpl_ref):
    g = pl.program_id(0)

    @pl.when(g == 0)
    def _init_template():
        k = jax.lax.broadcasted_iota(jnp.int32, (1, _E), 1)
        i = jnp.right_shift(k + jnp.right_shift(k, 8) + 1, 8)
        j = k - ((i << 8) - i)
        src = i
        dst = j + (j >= i).astype(jnp.int32)
        for q in range(_G):
            tmpl_ref[0:1, q * _E:(q + 1) * _E] = src + q * _N
            tmpl_ref[1:2, q * _E:(q + 1) * _E] = dst + q * _N

    edges_ref[...] = tmpl_ref[...] + g * (_G * _N)

    rows = jax.lax.broadcasted_iota(jnp.int32, (_G * _N, 1), 0)
    p = jnp.bitwise_and(rows, _N - 1)
    hr = jnp.right_shift(p, 4).astype(jnp.float32) * (1.0 / (_H - 1))
    wc = jnp.bitwise_and(p, _W - 1).astype(jnp.float32) * (1.0 / (_W - 1))
    nodes_ref[...] = jnp.concatenate([x_ref[...], hr, wc], axis=1)
    batch_ref[...] = g * _G + jnp.right_shift(rows, 8)


def kernel(x, y):
    x2d = x.reshape(_R, _C)
    edge_index, nodes, batch2 = pl.pallas_call(
        _build_kernel,
        grid=(_STEPS,),
        in_specs=[pl.BlockSpec((_G * _N, _C), lambda g: (g, 0))],
        out_specs=[
            pl.BlockSpec((2, _G * _E), lambda g: (0, g)),
            pl.BlockSpec((_G * _N, _C + 2), lambda g: (g, 0)),
            pl.BlockSpec((_G * _N, 1), lambda g: (g, 0)),
        ],
        out_shape=[
            jax.ShapeDtypeStruct((2, _B * _E), jnp.int32),
            jax.ShapeDtypeStruct((_R, _C + 2), jnp.float32),
            jax.ShapeDtypeStruct((_R, 1), jnp.int32),
        ],
        scratch_shapes=[pltpu.VMEM((2, _G * _E), jnp.int32)],
    )(x2d)
    batch_vec = batch2.reshape(_R)
    y_out = y.reshape(_B, -1)
    return nodes, edge_index, batch_vec, y_out


# speedup vs baseline: 1.1330x; 1.1330x over previous
import jax
import jax.numpy as jnp
from jax.experimental import pallas as pl
from jax.experimental.pallas import tpu as pltpu

_B, _H, _W, _C = 32, 16, 16, 64
_N = _H * _W
_R = _B * _N
_E = _N * (_N - 1)
_G = 4
_STEPS = _B // _G


def _build_kernel(x_ref, edges_ref, nodes_ref, batch_ref, tmpl_ref):
    g = pl.program_id(0)

    @pl.when(g == 0)
    def _init_template():
        k = jax.lax.broadcasted_iota(jnp.int32, (1, _E), 1)
        i = jnp.right_shift(k + jnp.right_shift(k, 8) + 1, 8)
        j = k - ((i << 8) - i)
        src = i
        dst = j + (j >= i).astype(jnp.int32)
        for q in range(_G):
            tmpl_ref[0:1, q * _E:(q + 1) * _E] = src + q * _N
            tmpl_ref[1:2, q * _E:(q + 1) * _E] = dst + q * _N

    edges_ref[...] = tmpl_ref[...] + g * (_G * _N)

    rows = jax.lax.broadcasted_iota(jnp.int32, (_G * _N, 1), 0)
    p = jnp.bitwise_and(rows, _N - 1)
    hr = jnp.right_shift(p, 4).astype(jnp.float32) * (1.0 / (_H - 1))
    wc = jnp.bitwise_and(p, _W - 1).astype(jnp.float32) * (1.0 / (_W - 1))
    nodes_ref[...] = jnp.concatenate([x_ref[...], hr, wc], axis=1)
    batch_ref[...] = g * _G + jnp.right_shift(rows, 8)


def kernel(x, y):
    x2d = x.reshape(_R, _C)
    edge_index, nodes, batch2 = pl.pallas_call(
        _build_kernel,
        grid=(_STEPS,),
        in_specs=[pl.BlockSpec((_G * _N, _C), lambda g: (g, 0))],
        out_specs=[
            pl.BlockSpec((2, _G * _E), lambda g: (0, g)),
            pl.BlockSpec((_G * _N, _C + 2), lambda g: (g, 0)),
            pl.BlockSpec((_G * _N, 1), lambda g: (g, 0)),
        ],
        out_shape=[
            jax.ShapeDtypeStruct((2, _B * _E), jnp.int32),
            jax.ShapeDtypeStruct((_R, _C + 2), jnp.float32),
            jax.ShapeDtypeStruct((_R, 1), jnp.int32),
        ],
        scratch_shapes=[pltpu.VMEM((2, _G * _E), jnp.int32)],
    )(x2d)
    batch_vec = batch2.reshape(_R)
    y_out = y.reshape(_B, -1)
    return nodes, edge_index, batch_vec, y_out


# speedup vs baseline: 1.2787x; 1.1286x over previous
import jax
import jax.numpy as jnp
from jax.experimental import pallas as pl
from jax.experimental.pallas import tpu as pltpu

_B, _H, _W, _C = 32, 16, 16, 64
_N = _H * _W
_R = _B * _N
_E = _N * (_N - 1)
_G = 4
_STEPS = _B // _G


def _build_kernel(x_ref, edges_ref, nodes_ref, batch_ref, tmpl_ref):
    g = pl.program_id(0)

    @pl.when(g == 0)
    def _init_template():
        k = jax.lax.broadcasted_iota(jnp.int32, (1, _E), 1)
        i = jnp.right_shift(k + jnp.right_shift(k, 8) + 1, 8)
        j = k - ((i << 8) - i)
        src = i
        dst = j + (j >= i).astype(jnp.int32)
        for q in range(_G):
            tmpl_ref[0:1, q * _E:(q + 1) * _E] = src + q * _N
            tmpl_ref[1:2, q * _E:(q + 1) * _E] = dst + q * _N

    edges_ref[...] = tmpl_ref[...] + g * (_G * _N)

    rows = jax.lax.broadcasted_iota(jnp.int32, (_G * _N, 1), 0)
    p = jnp.bitwise_and(rows, _N - 1)
    hr = jnp.right_shift(p, 4).astype(jnp.float32) * (1.0 / (_H - 1))
    wc = jnp.bitwise_and(p, _W - 1).astype(jnp.float32) * (1.0 / (_W - 1))
    nodes_ref[...] = jnp.concatenate([x_ref[...], hr, wc], axis=1)
    brow = jax.lax.broadcasted_iota(jnp.int32, (_G * _N // 128, 128), 0)
    batch_ref[...] = g * _G + jnp.right_shift(brow, 1)


def kernel(x, y):
    x2d = x.reshape(_R, _C)
    edge_index, nodes, batch2 = pl.pallas_call(
        _build_kernel,
        grid=(_STEPS,),
        in_specs=[pl.BlockSpec((_G * _N, _C), lambda g: (g, 0))],
        out_specs=[
            pl.BlockSpec((2, _G * _E), lambda g: (0, g)),
            pl.BlockSpec((_G * _N, _C + 2), lambda g: (g, 0)),
            pl.BlockSpec((_G * _N // 128, 128), lambda g: (g, 0)),
        ],
        out_shape=[
            jax.ShapeDtypeStruct((2, _B * _E), jnp.int32),
            jax.ShapeDtypeStruct((_R, _C + 2), jnp.float32),
            jax.ShapeDtypeStruct((_R // 128, 128), jnp.int32),
        ],
        scratch_shapes=[pltpu.VMEM((2, _G * _E), jnp.int32)],
    )(x2d)
    batch_vec = batch2.reshape(_R)
    y_out = y.reshape(_B, -1)
    return nodes, edge_index, batch_vec, y_out


# speedup vs baseline: 1.8140x; 1.4187x over previous
import jax
import jax.numpy as jnp
from jax.experimental import pallas as pl
from jax.experimental.pallas import tpu as pltpu

_B, _H, _W, _C = 32, 16, 16, 64
_N = _H * _W
_R = _B * _N
_E = _N * (_N - 1)
_G = 4
_STEPS = _B // _G


def _build_kernel(x_ref, edges_ref, nodes_ref, batch_ref, tmpl_ref):
    g = pl.program_id(0)

    @pl.when(g == 0)
    def _init_template():
        k = jax.lax.broadcasted_iota(jnp.int32, (1, _E), 1)
        i = jnp.right_shift(k + jnp.right_shift(k, 8) + 1, 8)
        j = k - ((i << 8) - i)
        src = i
        dst = j + (j >= i).astype(jnp.int32)
        for q in range(_G):
            tmpl_ref[0:1, q * _E:(q + 1) * _E] = src + q * _N
            tmpl_ref[1:2, q * _E:(q + 1) * _E] = dst + q * _N

    edges_ref[...] = tmpl_ref[...] + g * (_G * _N)

    cols = jax.lax.broadcasted_iota(jnp.int32, (1, _G * _N), 1)
    p = jnp.bitwise_and(cols, _N - 1)
    hr = jnp.right_shift(p, 4).astype(jnp.float32) * (1.0 / (_H - 1))
    wc = jnp.bitwise_and(p, _W - 1).astype(jnp.float32) * (1.0 / (_W - 1))
    nodes_ref[...] = jnp.concatenate(
        [jnp.transpose(x_ref[...], (1, 0)), hr, wc], axis=0)
    brow = jax.lax.broadcasted_iota(jnp.int32, (_G * _N // 128, 128), 0)
    batch_ref[...] = g * _G + jnp.right_shift(brow, 1)


def kernel(x, y):
    x2d = x.reshape(_R, _C)
    edge_index, nodes_t, batch2 = pl.pallas_call(
        _build_kernel,
        grid=(_STEPS,),
        in_specs=[pl.BlockSpec((_G * _N, _C), lambda g: (g, 0))],
        out_specs=[
            pl.BlockSpec((2, _G * _E), lambda g: (0, g)),
            pl.BlockSpec((_C + 2, _G * _N), lambda g: (0, g)),
            pl.BlockSpec((_G * _N // 128, 128), lambda g: (g, 0)),
        ],
        out_shape=[
            jax.ShapeDtypeStruct((2, _B * _E), jnp.int32),
            jax.ShapeDtypeStruct((_C + 2, _R), jnp.float32),
            jax.ShapeDtypeStruct((_R // 128, 128), jnp.int32),
        ],
        scratch_shapes=[pltpu.VMEM((2, _G * _E), jnp.int32)],
    )(x2d)
    nodes = nodes_t.T
    batch_vec = batch2.reshape(_R)
    y_out = y.reshape(_B, -1)
    return nodes, edge_index, batch_vec, y_out
